# Initial kernel scaffold; baseline (speedup 1.0000x reference)
import jax
import jax.numpy as jnp
from jax.experimental import pallas as pl


def kernel(x, table):
    raise NotImplementedError("write your pallas kernel here")


# speedup vs baseline: 1.5639x; 1.5639x over previous
import functools

import jax
import jax.numpy as jnp
from jax import lax
from jax.experimental import pallas as pl
from jax.experimental.pallas import tpu as pltpu
from jax.experimental.pallas import tpu_sc as plsc

NC = 2
NS = 16
NW = NC * NS


@functools.lru_cache(maxsize=None)
def _make_gather(V, D, B):
    assert B % NW == 0
    b_per_w = B // NW
    CH = 832
    assert b_per_w % CH == 0
    nchunk = b_per_w // CH
    mesh = plsc.VectorSubcoreMesh(core_axis_name="c", subcore_axis_name="s")

    @functools.partial(
        pl.kernel,
        mesh=mesh,
        out_type=jax.ShapeDtypeStruct((B, D), jnp.float32),
        scratch_types=[
            pltpu.VMEM((b_per_w,), jnp.int32),
            pltpu.VMEM((2, CH, D), jnp.float32),
            pltpu.SemaphoreType.DMA,
            pltpu.SemaphoreType.DMA,
            pltpu.SemaphoreType.DMA,
        ],
        compiler_params=pltpu.CompilerParams(use_tc_tiling_on_sc=False),
    )
    def gather_kernel(table_hbm, idx_hbm, out_hbm, idx_v, rows_v, gsem, psem0, psem1):
        wid = lax.axis_index("s") * NC + lax.axis_index("c")
        base = wid * b_per_w
        pltpu.sync_copy(idx_hbm.at[pl.ds(base, b_per_w)], idx_v)

        psems = (psem0, psem1)

        def start_gather(g):
            return pltpu.async_copy(
                table_hbm.at[idx_v.at[pl.ds(g * CH, CH)]],
                rows_v.at[g % 2],
                gsem,
            )

        puts = [None] * nchunk
        gathers = [None] * (nchunk + 1)
        gathers[0] = start_gather(0)
        for g in range(nchunk):
            gathers[g].wait()
            puts[g] = pltpu.async_copy(
                rows_v.at[g % 2],
                out_hbm.at[pl.ds(base + g * CH, CH)],
                psems[g % 2],
            )
            if g + 1 < nchunk:
                if g >= 1:
                    puts[g - 1].wait()
                gathers[g + 1] = start_gather(g + 1)
        puts[nchunk - 1].wait()
        if nchunk >= 2:
            puts[nchunk - 2].wait()

    return gather_kernel


def kernel(x, table):
    B0, B1 = x.shape
    V, D = table.shape
    B = B0 * B1
    flat = x.reshape(B).astype(jnp.int32)
    out = _make_gather(V, D, B)(table, flat)
    return out.reshape(B0, B1, D)
